# speedup vs baseline: 6.5415x; 6.5415x over previous
import functools

import jax
import jax.numpy as jnp
from jax import lax
from jax.experimental import pallas as pl
from jax.experimental.pallas import tpu as pltpu
from jax.experimental.pallas import tpu_sc as plsc

NN = 10000
EE = 320000
DD = 128
GG = 64
CC = 10

NC = 2
NS = 16
NW = NC * NS
EPT = EE // NW
CH = 80
NCHUNK = EPT // CH
N_PAD = 10240
RPT = N_PAD // NS

BLK = 400
NBLK = NN // BLK


def _segment_sum_sc(feat, src3, dst3, zeros):
    mesh = plsc.VectorSubcoreMesh(core_axis_name="c", subcore_axis_name="s")

    @functools.partial(
        pl.kernel,
        out_type=jax.ShapeDtypeStruct((NC, N_PAD, DD), jnp.float32),
        mesh=mesh,
        scratch_types=[
            pltpu.VMEM((NCHUNK, CH), jnp.int32),
            pltpu.VMEM((NCHUNK, CH), jnp.int32),
            pltpu.VMEM((CH, DD), jnp.float32),
            pltpu.VMEM_SHARED((N_PAD, DD), jnp.float32),
            pltpu.SemaphoreType.DMA,
        ],
    )
    def sc_kernel(x_hbm, src_hbm, dst_hbm, zero_hbm, out_hbm,
                  src_v, dst_v, rows_v, acc_sh, sem):
        cid = lax.axis_index("c")
        sid = lax.axis_index("s")
        wid = cid * NS + sid
        pltpu.sync_copy(zero_hbm, acc_sh.at[pl.ds(sid * RPT, RPT)])
        pltpu.sync_copy(src_hbm.at[wid], src_v)
        pltpu.sync_copy(dst_hbm.at[wid], dst_v)
        plsc.subcore_barrier()

        @pl.loop(0, NCHUNK)
        def _(j):
            pltpu.async_copy(x_hbm.at[src_v.at[j]], rows_v, sem).wait()
            pltpu.sync_copy(rows_v, acc_sh.at[dst_v.at[j]], add=True)

        plsc.subcore_barrier()
        pltpu.sync_copy(acc_sh.at[pl.ds(sid * RPT, RPT)],
                        out_hbm.at[cid, pl.ds(sid * RPT, RPT)])

    return sc_kernel(feat, src3, dst3, zeros)


def _mlp1_tc(x, parts, W1, b1, W2, b2):
    def body(x_ref, p0_ref, p1_ref, w1_ref, b1_ref, w2_ref, b2_ref, o_ref):
        u = x_ref[...] + p0_ref[0] + p1_ref[0]
        t = jnp.dot(u, w1_ref[...], preferred_element_type=jnp.float32)
        t = jnp.maximum(t + b1_ref[...], 0.0)
        h = jnp.dot(t, w2_ref[...], preferred_element_type=jnp.float32)
        o_ref[...] = jnp.maximum(h + b2_ref[...], 0.0)

    return pl.pallas_call(
        body,
        grid=(NBLK,),
        in_specs=[
            pl.BlockSpec((BLK, DD), lambda i: (i, 0)),
            pl.BlockSpec((1, BLK, DD), lambda i: (0, i, 0)),
            pl.BlockSpec((1, BLK, DD), lambda i: (1, i, 0)),
            pl.BlockSpec((DD, DD), lambda i: (0, 0)),
            pl.BlockSpec((1, DD), lambda i: (0, 0)),
            pl.BlockSpec((DD, DD), lambda i: (0, 0)),
            pl.BlockSpec((1, DD), lambda i: (0, 0)),
        ],
        out_specs=pl.BlockSpec((BLK, DD), lambda i: (i, 0)),
        out_shape=jax.ShapeDtypeStruct((NN, DD), jnp.float32),
    )(x, parts, parts, W1, b1.reshape(1, DD), W2, b2.reshape(1, DD))


def _mlp2_pool_tc(h, parts, batch2d, W3, b3, W4, b4, Wl1, bl1, Wl2, bl2):
    def body(h_ref, p0_ref, p1_ref, b_ref, w3_ref, b3_ref, w4_ref, b4_ref,
             wl1_ref, bl1_ref, wl2_ref, bl2_ref, o_ref, acc_ref, cnt_ref):
        i = pl.program_id(0)

        @pl.when(i == 0)
        def _():
            acc_ref[...] = jnp.zeros_like(acc_ref)
            cnt_ref[...] = jnp.zeros_like(cnt_ref)

        u = h_ref[...] + p0_ref[0] + p1_ref[0]
        t = jnp.dot(u, w3_ref[...], preferred_element_type=jnp.float32)
        t = jnp.maximum(t + b3_ref[...], 0.0)
        h2 = jnp.dot(t, w4_ref[...], preferred_element_type=jnp.float32)
        h2 = h2 + b4_ref[...]

        gids = lax.broadcasted_iota(jnp.int32, (1, GG), 1)
        onehot = (b_ref[...] == gids).astype(jnp.float32)
        acc_ref[...] += lax.dot_general(
            onehot, h2, (((0,), (0,)), ((), ())),
            preferred_element_type=jnp.float32)
        cnt_ref[...] += lax.dot_general(
            onehot, jnp.ones((BLK, DD), jnp.float32), (((0,), (0,)), ((), ())),
            preferred_element_type=jnp.float32)

        @pl.when(i == NBLK - 1)
        def _():
            pooled = acc_ref[...] / jnp.maximum(cnt_ref[...], 1.0)
            r = jnp.dot(pooled, wl1_ref[...],
                        preferred_element_type=jnp.float32) + bl1_ref[...]
            o_ref[...] = jnp.dot(r, wl2_ref[...],
                                 preferred_element_type=jnp.float32) + bl2_ref[...]

    return pl.pallas_call(
        body,
        grid=(NBLK,),
        in_specs=[
            pl.BlockSpec((BLK, DD), lambda i: (i, 0)),
            pl.BlockSpec((1, BLK, DD), lambda i: (0, i, 0)),
            pl.BlockSpec((1, BLK, DD), lambda i: (1, i, 0)),
            pl.BlockSpec((BLK, 1), lambda i: (i, 0)),
            pl.BlockSpec((DD, DD), lambda i: (0, 0)),
            pl.BlockSpec((1, DD), lambda i: (0, 0)),
            pl.BlockSpec((DD, DD), lambda i: (0, 0)),
            pl.BlockSpec((1, DD), lambda i: (0, 0)),
            pl.BlockSpec((DD, DD // 2), lambda i: (0, 0)),
            pl.BlockSpec((1, DD // 2), lambda i: (0, 0)),
            pl.BlockSpec((DD // 2, CC), lambda i: (0, 0)),
            pl.BlockSpec((1, CC), lambda i: (0, 0)),
        ],
        out_specs=pl.BlockSpec((GG, CC), lambda i: (0, 0)),
        out_shape=jax.ShapeDtypeStruct((GG, CC), jnp.float32),
        scratch_shapes=[
            pltpu.VMEM((GG, DD), jnp.float32),
            pltpu.VMEM((GG, DD), jnp.float32),
        ],
    )(h, parts, parts, batch2d, W3, b3.reshape(1, DD), W4, b4.reshape(1, DD),
      Wl1, bl1.reshape(1, DD // 2), Wl2, bl2.reshape(1, CC))


def kernel(x, edge_index, batch, W1, b1, W2, b2, W3, b3, W4, b4,
           Wl1, bl1, Wl2, bl2):
    src3 = edge_index[0].reshape(NW, NCHUNK, CH)
    dst3 = edge_index[1].reshape(NW, NCHUNK, CH)
    zeros = jnp.zeros((RPT, DD), jnp.float32)
    batch2d = batch.reshape(NN, 1)

    parts1 = _segment_sum_sc(x, src3, dst3, zeros)
    h = _mlp1_tc(x, parts1, W1, b1, W2, b2)
    parts2 = _segment_sum_sc(h, src3, dst3, zeros)
    return _mlp2_pool_tc(h, parts2, batch2d, W3, b3, W4, b4, Wl1, bl1, Wl2, bl2)
